# Initial kernel scaffold; baseline (speedup 1.0000x reference)
import jax
import jax.numpy as jnp
from jax.experimental import pallas as pl


def kernel(x, edge_index, node_features, W1_l, b1, W1_r, W2_l, b2, W2_r):
    raise NotImplementedError("write your pallas kernel here")


# speedup vs baseline: 4.2172x; 4.2172x over previous
import functools

import jax
import jax.numpy as jnp
from jax import lax
from jax.experimental import pallas as pl
from jax.experimental.pallas import tpu as pltpu
from jax.experimental.pallas import tpu_sc as plsc

NC = 2
NS = 16
NW = NC * NS
CHUNK = 128
CB = 128


def _slab_chunks(rps):
  chunks = []
  off = 0
  while off < rps:
    sz = min(CHUNK, rps - off)
    chunks.append((off, sz))
    off += sz
  return chunks


def _fill(buf, rows_n, cols, value):
  vec = jnp.full((16,), value, jnp.float32)

  @pl.loop(0, rows_n)
  def _(i):
    for j in range(cols // 16):
      buf[i, pl.ds(j * 16, 16)] = vec


def _make_sc_gather_test(n_rows, d):
  mesh = plsc.VectorSubcoreMesh(core_axis_name="c", subcore_axis_name="s")

  n_pad = -(-(n_rows + 1) // (NS * 8)) * (NS * 8)
  rps = n_pad // NS
  chunks = _slab_chunks(rps)

  @functools.partial(
      pl.kernel, mesh=mesh,
      out_type=jax.ShapeDtypeStruct((NW * CHUNK, d), jnp.float32),
      scratch_types=(
          pltpu.VMEM((CHUNK,), jnp.int32),
          pltpu.VMEM((CHUNK,), jnp.int32),
          pltpu.VMEM((CHUNK, d), jnp.float32),
          pltpu.VMEM_SHARED((n_pad, d), jnp.float32),
          pltpu.SemaphoreType.DMA,
      ),
  )
  def k(feats, srci, dsti, out_hbm, idx_v, idx_d, rows_v, sh, sem):
    s = lax.axis_index("s")
    wid = s * NC + lax.axis_index("c")
    base = s * rps
    _fill(rows_v, CHUNK, d, 0.0)
    for off, sz in chunks:
      pltpu.sync_copy(rows_v.at[pl.ds(0, sz)], sh.at[pl.ds(base + off, sz)])
    plsc.subcore_barrier()

    @pl.loop(0, 2)
    def _(kk):
      pltpu.sync_copy(srci.at[wid, kk], idx_v)
      pltpu.sync_copy(dsti.at[wid, kk], idx_d)
      pltpu.async_copy(feats.at[idx_v], rows_v, sem).wait()
      pltpu.sync_copy(rows_v, sh.at[idx_d], add=True)

    plsc.subcore_barrier()
    pltpu.sync_copy(sh.at[pl.ds(s * CHUNK, CHUNK)], rows_v)
    pltpu.sync_copy(rows_v, out_hbm.at[pl.ds(wid * CHUNK, CHUNK)])

  return k


def _make_sc_cnt(n_pad, k_steps):
  rps = n_pad // NS
  chunks = _slab_chunks(rps)
  mesh = plsc.VectorSubcoreMesh(core_axis_name="c", subcore_axis_name="s")

  @functools.partial(
      pl.kernel, mesh=mesh,
      out_type=jax.ShapeDtypeStruct((NC, n_pad, CB), jnp.float32),
      scratch_types=(
          pltpu.VMEM_SHARED((n_pad, CB), jnp.float32),
          pltpu.VMEM((CHUNK,), jnp.int32),
          pltpu.VMEM((CHUNK, CB), jnp.float32),
      ),
  )
  def k(dsti, cntp, sh_cnt, idx_d, ones_v):
    c = lax.axis_index("c")
    s = lax.axis_index("s")
    wid = s * NC + c
    base = s * rps

    _fill(ones_v, CHUNK, CB, 0.0)
    for off, sz in chunks:
      pltpu.sync_copy(ones_v.at[pl.ds(0, sz)], sh_cnt.at[pl.ds(base + off, sz)])
    _fill(ones_v, CHUNK, CB, 1.0)
    plsc.subcore_barrier()

    @pl.loop(0, k_steps)
    def _(kk):
      pltpu.sync_copy(dsti.at[wid, kk], idx_d)
      pltpu.sync_copy(ones_v, sh_cnt.at[idx_d], add=True)

    plsc.subcore_barrier()
    for off, sz in chunks:
      pltpu.sync_copy(sh_cnt.at[pl.ds(base + off, sz)], ones_v.at[pl.ds(0, sz)])
      pltpu.sync_copy(ones_v.at[pl.ds(0, sz)], cntp.at[c, pl.ds(base + off, sz)])

  return k


def _make_sc_agg(n_pad, d, k_steps, with_cnt):
  rps = n_pad // NS
  chunks = _slab_chunks(rps)

  out_type = [jax.ShapeDtypeStruct((NC, n_pad, d), jnp.float32)]
  scratch = [
      pltpu.VMEM_SHARED((n_pad, d), jnp.float32),
      pltpu.VMEM((CHUNK,), jnp.int32),
      pltpu.VMEM((CHUNK,), jnp.int32),
      pltpu.VMEM((CHUNK, d), jnp.float32),
      pltpu.SemaphoreType.DMA,
  ]
  if with_cnt:
    out_type.append(jax.ShapeDtypeStruct((NC, n_pad, CB), jnp.float32))
    scratch.append(pltpu.VMEM_SHARED((n_pad, CB), jnp.float32))
    scratch.append(pltpu.VMEM((CHUNK, CB), jnp.float32))

  mesh = plsc.VectorSubcoreMesh(core_axis_name="c", subcore_axis_name="s")

  def body(feats, srci, dsti, part, cntp,
           sh_agg, idx_s, idx_d, rows, sem, sh_cnt=None, ones_v=None):
    c = lax.axis_index("c")
    s = lax.axis_index("s")
    wid = s * NC + c
    base = s * rps

    _fill(rows, CHUNK, d, 0.0)
    for off, sz in chunks:
      pltpu.sync_copy(rows.at[pl.ds(0, sz)], sh_agg.at[pl.ds(base + off, sz)])
    if with_cnt:
      _fill(ones_v, CHUNK, CB, 0.0)
      for off, sz in chunks:
        pltpu.sync_copy(ones_v.at[pl.ds(0, sz)],
                        sh_cnt.at[pl.ds(base + off, sz)])
      _fill(ones_v, CHUNK, CB, 1.0)
    plsc.subcore_barrier()

    @pl.loop(0, k_steps)
    def _(k):
      pltpu.sync_copy(srci.at[wid, k], idx_s)
      pltpu.sync_copy(dsti.at[wid, k], idx_d)
      pltpu.async_copy(feats.at[idx_s], rows, sem).wait()
      pltpu.sync_copy(rows, sh_agg.at[idx_d], add=True)
      if with_cnt:
        pltpu.sync_copy(ones_v, sh_cnt.at[idx_d], add=True)

    plsc.subcore_barrier()

    for off, sz in chunks:
      pltpu.sync_copy(sh_agg.at[pl.ds(base + off, sz)], rows.at[pl.ds(0, sz)])
      pltpu.sync_copy(rows.at[pl.ds(0, sz)], part.at[c, pl.ds(base + off, sz)])
    if with_cnt:
      for off, sz in chunks:
        pltpu.sync_copy(sh_cnt.at[pl.ds(base + off, sz)],
                        ones_v.at[pl.ds(0, sz)])
        pltpu.sync_copy(ones_v.at[pl.ds(0, sz)],
                        cntp.at[c, pl.ds(base + off, sz)])

  if with_cnt:
    def fn(feats, srci, dsti, part, cntp,
           sh_agg, idx_s, idx_d, rows, sem, sh_cnt, ones_v):
      body(feats, srci, dsti, part, cntp,
           sh_agg, idx_s, idx_d, rows, sem, sh_cnt, ones_v)
  else:
    def fn(feats, srci, dsti, part,
           sh_agg, idx_s, idx_d, rows, sem):
      body(feats, srci, dsti, part, None,
           sh_agg, idx_s, idx_d, rows, sem)

  return pl.kernel(fn, out_type=tuple(out_type), mesh=mesh,
                   scratch_types=tuple(scratch))


def _tc_dense(part, cnt3, x, w_l, w_r, b, *, relu, n, d):
  rblk = 1000
  grid = (n // rblk,)

  def body(part_ref, cnt_ref, x_ref, wl_ref, wr_ref, b_ref, o_ref):
    p = part_ref[0] + part_ref[1]
    cn = cnt_ref[0, :, 0:1] + cnt_ref[1, :, 0:1]
    mean = p / jnp.maximum(cn, 1.0)
    acc = (jnp.dot(mean, wl_ref[...], preferred_element_type=jnp.float32)
           + jnp.dot(x_ref[...], wr_ref[...], preferred_element_type=jnp.float32)
           + b_ref[...])
    if relu:
      acc = jnp.maximum(acc, 0.0)
    o_ref[...] = acc

  return pl.pallas_call(
      body,
      grid=grid,
      in_specs=[
          pl.BlockSpec((NC, rblk, d), lambda i: (0, i, 0)),
          pl.BlockSpec((NC, rblk, CB), lambda i: (0, i, 0)),
          pl.BlockSpec((rblk, d), lambda i: (i, 0)),
          pl.BlockSpec((d, d), lambda i: (0, 0)),
          pl.BlockSpec((d, d), lambda i: (0, 0)),
          pl.BlockSpec((1, d), lambda i: (0, 0)),
      ],
      out_specs=pl.BlockSpec((rblk, d), lambda i: (i, 0)),
      out_shape=jax.ShapeDtypeStruct((n, d), jnp.float32),
  )(part, cnt3, x, w_l, w_r, b)


def kernel(x, edge_index, node_features, W1_l, b1, W1_r, W2_l, b2, W2_r):
  del x
  n, d = node_features.shape
  e = edge_index.shape[1]

  n_pad = -(-(n + 1) // (NS * 8)) * (NS * 8)
  k_steps = -(-e // (NW * CHUNK))
  e_pad = NW * k_steps * CHUNK

  src = edge_index[0]
  dst = edge_index[1]
  pad = e_pad - e
  srcp = jnp.concatenate([src, jnp.zeros((pad,), jnp.int32)]).reshape(
      NW, k_steps, CHUNK)
  dstp = jnp.concatenate([dst, jnp.full((pad,), n, jnp.int32)]).reshape(
      NW, k_steps, CHUNK)

  agg = _make_sc_agg(n_pad, d, k_steps, with_cnt=False)

  cntp = _make_sc_cnt(n_pad, k_steps)(dstp)

  part1 = agg(node_features, srcp, dstp)
  if isinstance(part1, (tuple, list)):
    part1 = part1[0]
  b1r = b1.reshape(1, d)
  b2r = b2.reshape(1, d)
  h1 = _tc_dense(part1, cntp, node_features, W1_l, W1_r, b1r,
                 relu=True, n=n, d=d)
  part2 = agg(h1, srcp, dstp)
  if isinstance(part2, (tuple, list)):
    part2 = part2[0]
  out = _tc_dense(part2, cntp, h1, W2_l, W2_r, b2r, relu=False, n=n, d=d)
  return out
